# Initial kernel scaffold; baseline (speedup 1.0000x reference)
import jax
import jax.numpy as jnp
from jax.experimental import pallas as pl


def kernel(src_idx, dst_idx, src_feat, dst_feat):
    raise NotImplementedError("write your pallas kernel here")


# speedup vs baseline: 3.5579x; 3.5579x over previous
import functools

import jax
import jax.numpy as jnp
from jax import lax
from jax.experimental import pallas as pl
from jax.experimental.pallas import tpu as pltpu
from jax.experimental.pallas import tpu_sc as plsc

N_NODES = 10000
N_EDGES = 320000
D_FEAT = 128
NUM_CORES = 2
NUM_SUBCORES = 16
NUM_WORKERS = NUM_CORES * NUM_SUBCORES
EDGES_PER_WORKER = N_EDGES // NUM_WORKERS
CHUNK = 80
NUM_CHUNKS = EDGES_PER_WORKER // CHUNK


def kernel(src_idx, dst_idx, src_feat, dst_feat):
    mesh = plsc.VectorSubcoreMesh(core_axis_name="c", subcore_axis_name="s")

    @functools.partial(
        pl.kernel,
        mesh=mesh,
        out_type=jax.ShapeDtypeStruct((N_EDGES,), jnp.float32),
        compiler_params=pltpu.CompilerParams(needs_layout_passes=False),
        scratch_types=[
            pltpu.VMEM((CHUNK,), jnp.int32),
            pltpu.VMEM((CHUNK,), jnp.int32),
            pltpu.VMEM((CHUNK, D_FEAT), jnp.float32),
            pltpu.VMEM((CHUNK, D_FEAT), jnp.float32),
            pltpu.VMEM((CHUNK,), jnp.float32),
            pltpu.VMEM((256,), jnp.float32),
            pltpu.SemaphoreType.DMA,
            pltpu.SemaphoreType.DMA,
        ],
    )
    def k(sidx_hbm, didx_hbm, sfeat_hbm, dfeat_hbm, out_hbm,
          sidx_v, didx_v, srows, drows, outb, pbuf, sem_s, sem_d):
        wid = lax.axis_index("s") * NUM_CORES + lax.axis_index("c")
        base0 = wid * EDGES_PER_WORKER

        def chunk_body(ci, _):
            base = base0 + ci * CHUNK
            pltpu.sync_copy(sidx_hbm.at[pl.ds(base, CHUNK)], sidx_v)
            pltpu.sync_copy(didx_hbm.at[pl.ds(base, CHUNK)], didx_v)
            cp_s = pltpu.async_copy(sfeat_hbm.at[sidx_v], srows, sem_s)
            cp_d = pltpu.async_copy(dfeat_hbm.at[didx_v], drows, sem_d)
            cp_s.wait()
            cp_d.wait()

            colbase = lax.iota(jnp.int32, 16) * 16

            def group_body(g, _):
                e0 = g * 16
                for j in range(16):
                    e = e0 + j
                    acc = srows[e, pl.ds(0, 16)] * drows[e, pl.ds(0, 16)]
                    for kk in range(1, D_FEAT // 16):
                        acc = acc + (srows[e, pl.ds(kk * 16, 16)]
                                     * drows[e, pl.ds(kk * 16, 16)])
                    pbuf[pl.ds(j * 16, 16)] = acc
                tot = plsc.load_gather(pbuf, [colbase])
                for j in range(1, 16):
                    tot = tot + plsc.load_gather(pbuf, [colbase + j])
                outb[pl.ds(e0, 16)] = tot
                return 0

            lax.fori_loop(0, CHUNK // 16, group_body, 0)
            pltpu.sync_copy(outb, out_hbm.at[pl.ds(base, CHUNK)])
            return 0

        lax.fori_loop(0, NUM_CHUNKS, chunk_body, 0)

    return k(src_idx, dst_idx, src_feat, dst_feat)


# speedup vs baseline: 7.6327x; 2.1453x over previous
import functools

import jax
import jax.numpy as jnp
from jax import lax
from jax.experimental import pallas as pl
from jax.experimental.pallas import tpu as pltpu
from jax.experimental.pallas import tpu_sc as plsc

N_NODES = 10000
N_EDGES = 320000
D_FEAT = 128
NUM_CORES = 2
NUM_SUBCORES = 16
NUM_WORKERS = NUM_CORES * NUM_SUBCORES
EDGES_PER_WORKER = N_EDGES // NUM_WORKERS
CHUNK = 80
NUM_CHUNKS = EDGES_PER_WORKER // CHUNK


def kernel(src_idx, dst_idx, src_feat, dst_feat):
    mesh = plsc.VectorSubcoreMesh(core_axis_name="c", subcore_axis_name="s")

    @functools.partial(
        pl.kernel,
        mesh=mesh,
        out_type=jax.ShapeDtypeStruct((N_EDGES,), jnp.float32),
        compiler_params=pltpu.CompilerParams(needs_layout_passes=False),
        scratch_types=[
            pltpu.VMEM((EDGES_PER_WORKER,), jnp.int32),
            pltpu.VMEM((EDGES_PER_WORKER,), jnp.int32),
            pltpu.VMEM((CHUNK, D_FEAT), jnp.float32),
            pltpu.VMEM((CHUNK, D_FEAT), jnp.float32),
            pltpu.VMEM((CHUNK, D_FEAT), jnp.float32),
            pltpu.VMEM((CHUNK, D_FEAT), jnp.float32),
            pltpu.VMEM((EDGES_PER_WORKER,), jnp.float32),
            pltpu.VMEM((256,), jnp.float32),
            pltpu.SemaphoreType.DMA,
            pltpu.SemaphoreType.DMA,
            pltpu.SemaphoreType.DMA,
            pltpu.SemaphoreType.DMA,
        ],
    )
    def k(sidx_hbm, didx_hbm, sfeat_hbm, dfeat_hbm, out_hbm,
          sidx_v, didx_v, srows0, drows0, srows1, drows1, out_v, pbuf,
          sem_s0, sem_d0, sem_s1, sem_d1):
        wid = lax.axis_index("s") * NUM_CORES + lax.axis_index("c")
        base0 = wid * EDGES_PER_WORKER
        pltpu.sync_copy(sidx_hbm.at[pl.ds(base0, EDGES_PER_WORKER)], sidx_v)
        pltpu.sync_copy(didx_hbm.at[pl.ds(base0, EDGES_PER_WORKER)], didx_v)

        sbufs = (srows0, srows1)
        dbufs = (drows0, drows1)
        sems = ((sem_s0, sem_d0), (sem_s1, sem_d1))

        def gather_descs(ci, b):
            off = ci * CHUNK
            return (
                pltpu.make_async_copy(
                    sfeat_hbm.at[sidx_v.at[pl.ds(off, CHUNK)]],
                    sbufs[b], sems[b][0]),
                pltpu.make_async_copy(
                    dfeat_hbm.at[didx_v.at[pl.ds(off, CHUNK)]],
                    dbufs[b], sems[b][1]),
            )

        def gather_start(ci, b):
            for cp in gather_descs(ci, b):
                cp.start()

        def gather_wait(ci, b):
            for cp in gather_descs(ci, b):
                cp.wait()

        colbase = lax.iota(jnp.int32, 16) * 16

        def compute_chunk(ci, b):
            sbuf, dbuf = sbufs[b], dbufs[b]
            obase = ci * CHUNK

            def group_body(g, _):
                e0 = g * 16
                for j in range(16):
                    e = e0 + j
                    prods = [sbuf[e, pl.ds(kk * 16, 16)]
                             * dbuf[e, pl.ds(kk * 16, 16)]
                             for kk in range(D_FEAT // 16)]
                    while len(prods) > 1:
                        prods = [prods[i] + prods[i + 1]
                                 for i in range(0, len(prods), 2)]
                    pbuf[pl.ds(j * 16, 16)] = prods[0]
                tot = plsc.load_gather(pbuf, [colbase])
                for j in range(1, 16):
                    tot = tot + plsc.load_gather(pbuf, [colbase + j])
                out_v[pl.ds(obase + e0, 16)] = tot
                return 0

            lax.fori_loop(0, CHUNK // 16, group_body, 0)

        gather_start(0, 0)

        def pair_body(p, _):
            ci0 = 2 * p
            gather_start(ci0 + 1, 1)
            gather_wait(ci0, 0)
            compute_chunk(ci0, 0)
            gather_start(ci0 + 2, 0)
            gather_wait(ci0 + 1, 1)
            compute_chunk(ci0 + 1, 1)
            return 0

        lax.fori_loop(0, (NUM_CHUNKS - 1) // 2, pair_body, 0)
        gather_wait(NUM_CHUNKS - 1, 0)
        compute_chunk(NUM_CHUNKS - 1, 0)
        pltpu.sync_copy(out_v, out_hbm.at[pl.ds(base0, EDGES_PER_WORKER)])

    return k(src_idx, dst_idx, src_feat, dst_feat)
